# Initial kernel scaffold; baseline (speedup 1.0000x reference)
import jax
import jax.numpy as jnp
from jax.experimental import pallas as pl


def kernel(x, edge_index, W_in, b_in, g_in, be_in, Wl0, Wr0, bl0, g0, be0, se0_W1, se0_W2, Wl1, Wr1, bl1, g1, be1, se1_W1, se1_W2, Wl2, Wr2, bl2, g_f, be_f, W_out, b_out):
    raise NotImplementedError("write your pallas kernel here")


# speedup vs baseline: 8.3630x; 8.3630x over previous
import jax
import jax.numpy as jnp
from jax import lax
from jax.experimental import pallas as pl
from jax.experimental.pallas import tpu as pltpu
from jax.experimental.pallas import tpu_sc as plsc

NC, NS = 2, 16
NW = NC * NS
N_NODES = 10000
E_EDGES = 320000
CH = 125
NCT = E_EDGES // (NS * CH)
DCH, DNCT = 80, E_EDGES // (NW * 80)
N_PAD = 10112
ROWS_PER_TILE = N_PAD // NS


def _seg_body(src_hbm, dst_hbm, table_hbm, zeros_hbm, out_hbm,
              src_v, dst_v, rows0, rows1, acc, sem0, sem1):
    c = lax.axis_index("c")
    s = lax.axis_index("s")

    pltpu.sync_copy(src_hbm.at[s], src_v)
    pltpu.sync_copy(dst_hbm.at[s], dst_v)

    row0 = s * ROWS_PER_TILE
    pltpu.sync_copy(zeros_hbm.at[pl.ds(row0, ROWS_PER_TILE)],
                    acc.at[pl.ds(row0, ROWS_PER_TILE)])
    plsc.subcore_barrier()

    def gather_start(ci, buf, sem):
        pltpu.async_copy(table_hbm.at[c].at[src_v.at[ci]], buf, sem)

    def gather_wait(ci, buf, sem):
        pltpu.make_async_copy(table_hbm.at[c].at[src_v.at[ci]], buf, sem).wait()

    def scat(ci, buf):
        pltpu.sync_copy(buf, acc.at[dst_v.at[ci]], add=True)

    gather_start(0, rows0, sem0)

    def body(c2, _):
        c0 = 2 * c2
        gather_start(c0 + 1, rows1, sem1)
        gather_wait(c0, rows0, sem0)
        scat(c0, rows0)
        gather_start(c0 + 2, rows0, sem0)
        gather_wait(c0 + 1, rows1, sem1)
        scat(c0 + 1, rows1)
        return 0

    lax.fori_loop(0, (NCT - 2) // 2, body, 0)
    gather_start(NCT - 1, rows1, sem1)
    gather_wait(NCT - 2, rows0, sem0)
    scat(NCT - 2, rows0)
    gather_wait(NCT - 1, rows1, sem1)
    scat(NCT - 1, rows1)

    plsc.subcore_barrier()
    pltpu.sync_copy(acc.at[pl.ds(row0, ROWS_PER_TILE)],
                    out_hbm.at[c, pl.ds(row0, ROWS_PER_TILE)])


def _seg_halves(src3d, dst3d, table, zeros):
    mesh = plsc.VectorSubcoreMesh(core_axis_name="c", subcore_axis_name="s")
    f = pl.kernel(
        _seg_body,
        out_type=jax.ShapeDtypeStruct((NC, N_PAD, 64), jnp.float32),
        mesh=mesh,
        compiler_params=pltpu.CompilerParams(use_tc_tiling_on_sc=False),
        scratch_types=[
            pltpu.VMEM((NCT, CH), jnp.int32),
            pltpu.VMEM((NCT, CH), jnp.int32),
            pltpu.VMEM((CH, 64), jnp.float32),
            pltpu.VMEM((CH, 64), jnp.float32),
            pltpu.VMEM_SHARED((N_PAD, 64), jnp.float32),
            pltpu.SemaphoreType.DMA,
            pltpu.SemaphoreType.DMA,
        ],
    )
    return f(src3d, dst3d, table, zeros)


def _deg_body(dst_hbm, ones_hbm, zeros_hbm, out_hbm, dst_v, ones_v, dacc):
    c = lax.axis_index("c")
    s = lax.axis_index("s")
    wid = s * NC + c

    pltpu.sync_copy(dst_hbm.at[wid], dst_v)
    pltpu.sync_copy(ones_hbm, ones_v)

    row0 = s * ROWS_PER_TILE
    pltpu.sync_copy(zeros_hbm.at[pl.ds(row0, ROWS_PER_TILE)],
                    dacc.at[pl.ds(row0, ROWS_PER_TILE)])
    plsc.subcore_barrier()

    def body(ci, _):
        pltpu.sync_copy(ones_v, dacc.at[dst_v.at[ci]], add=True)
        return 0
    lax.fori_loop(0, DNCT, body, 0)

    plsc.subcore_barrier()
    pltpu.sync_copy(dacc.at[pl.ds(row0, ROWS_PER_TILE)],
                    out_hbm.at[c, pl.ds(row0, ROWS_PER_TILE)])


def _deg_partial(dst2d, ones, zeros16):
    mesh = plsc.VectorSubcoreMesh(core_axis_name="c", subcore_axis_name="s")
    f = pl.kernel(
        _deg_body,
        out_type=jax.ShapeDtypeStruct((NC, N_PAD, 16), jnp.float32),
        mesh=mesh,
        compiler_params=pltpu.CompilerParams(use_tc_tiling_on_sc=False),
        scratch_types=[
            pltpu.VMEM((DNCT, DCH), jnp.int32),
            pltpu.VMEM((DCH, 16), jnp.float32),
            pltpu.VMEM_SHARED((N_PAD, 16), jnp.float32),
        ],
    )
    return f(dst2d, ones, zeros16)


def _se(h, w1t, w2t):
    y = jnp.mean(h, axis=0, keepdims=True)
    a = jnp.maximum(jnp.dot(y, w1t, preferred_element_type=jnp.float32), 0.0)
    sg = jax.nn.sigmoid(jnp.dot(a, w2t, preferred_element_type=jnp.float32))
    return h * sg


def _bn(z, g, be):
    m = jnp.mean(z, axis=0, keepdims=True)
    v = jnp.mean((z - m) ** 2, axis=0, keepdims=True)
    return (z - m) / jnp.sqrt(v + 1e-5) * g + be


def _inproj_body(x_ref, w_ref, b_ref, g_ref, be_ref, tab_ref):
    z = jnp.dot(x_ref[...], w_ref[...], preferred_element_type=jnp.float32)
    z = z + b_ref[...]
    h = jnp.maximum(_bn(z, g_ref[...], be_ref[...]), 0.0)
    tab_ref[0] = h[:, :64]
    tab_ref[1] = h[:, 64:]


def _layer0_body(agg_ref, degp_ref, tab_ref, wla_ref, wlb_ref, wra_ref,
                 wrb_ref, bl_ref, g_ref, be_ref, w1_ref, w2_ref,
                 taba_ref, tabb_ref):
    degp = degp_ref[...]
    deg = jnp.maximum(degp[0, :N_NODES, 0:1] + degp[1, :N_NODES, 0:1], 1.0)
    agg = agg_ref[...]
    tab = tab_ref[...]
    z = (jnp.dot(agg[0, :N_NODES] / deg, wla_ref[...], preferred_element_type=jnp.float32)
         + jnp.dot(agg[1, :N_NODES] / deg, wlb_ref[...], preferred_element_type=jnp.float32)
         + bl_ref[...]
         + jnp.dot(tab[0], wra_ref[...], preferred_element_type=jnp.float32)
         + jnp.dot(tab[1], wrb_ref[...], preferred_element_type=jnp.float32))
    zn = _bn(z, g_ref[...], be_ref[...])
    h = jnp.maximum(_se(zn, w1_ref[...], w2_ref[...]), 0.0)
    taba_ref[0] = h[:, 0:64]
    taba_ref[1] = h[:, 64:128]
    tabb_ref[0] = h[:, 128:192]
    tabb_ref[1] = h[:, 192:256]


def _mean_body(agg_ref, degp_ref, mean_ref):
    degp = degp_ref[...]
    deg = jnp.maximum(degp[0, :N_NODES, 0:1] + degp[1, :N_NODES, 0:1], 1.0)
    agg = agg_ref[...]
    mean_ref[:, :64] = agg[0, :N_NODES] / deg
    mean_ref[:, 64:] = agg[1, :N_NODES] / deg


def _layer1_body(ma_ref, mb_ref, taba_ref, tabb_ref, wla_ref, wlb_ref,
                 wra_ref, wrb_ref, bl_ref, g_ref, be_ref, w1_ref, w2_ref,
                 h2a_ref, h2b_ref):
    taba = taba_ref[...]
    tabb = tabb_ref[...]
    ra = jnp.concatenate([taba[0], taba[1]], axis=1)
    rb = jnp.concatenate([tabb[0], tabb[1]], axis=1)
    z = (jnp.dot(ma_ref[...], wla_ref[...], preferred_element_type=jnp.float32)
         + jnp.dot(mb_ref[...], wlb_ref[...], preferred_element_type=jnp.float32)
         + bl_ref[...]
         + jnp.dot(ra, wra_ref[...], preferred_element_type=jnp.float32)
         + jnp.dot(rb, wrb_ref[...], preferred_element_type=jnp.float32))
    zn = _bn(z, g_ref[...], be_ref[...])
    h = jnp.maximum(_se(zn, w1_ref[...], w2_ref[...]), 0.0)
    h2a_ref[...] = h[:, :128] + ra
    h2b_ref[...] = h[:, 128:] + rb


def _pq_body(h2a_ref, h2b_ref, wl2_ref, wr2_ref, ptab_ref, q_ref):
    h2a = h2a_ref[...]
    h2b = h2b_ref[...]
    p = (jnp.dot(h2a, wl2_ref[:128], preferred_element_type=jnp.float32)
         + jnp.dot(h2b, wl2_ref[128:], preferred_element_type=jnp.float32))
    ptab_ref[0] = p[:, :64]
    ptab_ref[1] = p[:, 64:]
    q_ref[...] = (jnp.dot(h2a, wr2_ref[:128], preferred_element_type=jnp.float32)
                  + jnp.dot(h2b, wr2_ref[128:], preferred_element_type=jnp.float32))


def _final_body(agg_ref, degp_ref, q_ref, bl_ref, g_ref, be_ref,
                wo_ref, bo_ref, out_ref):
    degp = degp_ref[...]
    deg = jnp.maximum(degp[0, :N_NODES, 0:1] + degp[1, :N_NODES, 0:1], 1.0)
    agg = agg_ref[...]
    mean = jnp.concatenate([agg[0, :N_NODES], agg[1, :N_NODES]], axis=1) / deg
    z = mean + bl_ref[...] + q_ref[...]
    h = jnp.maximum(_bn(z, g_ref[...], be_ref[...]), 0.0)
    out_ref[...] = (jnp.dot(h, wo_ref[...], preferred_element_type=jnp.float32)
                    + bo_ref[...])


def _tc(body, out_shape, *args):
    return pl.pallas_call(
        body, out_shape=out_shape,
        compiler_params=pltpu.CompilerParams(vmem_limit_bytes=100 * 1024 * 1024),
    )(*args)


def kernel(x, edge_index, W_in, b_in, g_in, be_in, Wl0, Wr0, bl0, g0, be0,
           se0_W1, se0_W2, Wl1, Wr1, bl1, g1, be1, se1_W1, se1_W2,
           Wl2, Wr2, bl2, g_f, be_f, W_out, b_out):
    f32 = jnp.float32
    src3d = edge_index[0].reshape(NS, NCT, CH)
    dst3d = edge_index[1].reshape(NS, NCT, CH)
    dst_deg = edge_index[1].reshape(NW, DNCT, DCH)
    zeros = jnp.zeros((N_PAD, 64), f32)
    zeros16 = jnp.zeros((N_PAD, 16), f32)

    r1 = lambda a: a.reshape(1, -1)
    sds = jax.ShapeDtypeStruct
    tab_t = sds((NC, N_NODES, 64), f32)

    wl0t, wr0t = Wl0.T, Wr0.T
    wl1t, wr1t = Wl1.T, Wr1.T

    h0tab = _tc(_inproj_body, tab_t, x, W_in.T, r1(b_in), r1(g_in), r1(be_in))

    ones = jnp.ones((DCH, 16), f32)
    degp = _deg_partial(dst_deg, ones, zeros16)
    agg0 = _seg_halves(src3d, dst3d, h0tab, zeros)

    h1taba, h1tabb = _tc(
        _layer0_body, [tab_t, tab_t],
        agg0, degp, h0tab, wl0t[:64], wl0t[64:], wr0t[:64], wr0t[64:],
        r1(bl0), r1(g0), r1(be0), se0_W1.T, se0_W2.T)

    agg1a = _seg_halves(src3d, dst3d, h1taba, zeros)
    agg1b = _seg_halves(src3d, dst3d, h1tabb, zeros)
    m1a = _tc(_mean_body, sds((N_NODES, 128), f32), agg1a, degp)
    m1b = _tc(_mean_body, sds((N_NODES, 128), f32), agg1b, degp)

    h2a, h2b = _tc(
        _layer1_body, [sds((N_NODES, 128), f32), sds((N_NODES, 128), f32)],
        m1a, m1b, h1taba, h1tabb, wl1t[:128], wl1t[128:], wr1t[:128],
        wr1t[128:], r1(bl1), r1(g1), r1(be1), se1_W1.T, se1_W2.T)
    ptab, q = _tc(_pq_body, [tab_t, sds((N_NODES, 128), f32)],
                  h2a, h2b, Wl2.T, Wr2.T)

    agg2 = _seg_halves(src3d, dst3d, ptab, zeros)

    out = _tc(_final_body, sds((N_NODES, 128), f32),
              agg2, degp, q, r1(bl2), r1(g_f), r1(be_f), W_out.T, r1(b_out))
    return out


# speedup vs baseline: 9.1817x; 1.0979x over previous
import jax
import jax.numpy as jnp
from jax import lax
from jax.experimental import pallas as pl
from jax.experimental.pallas import tpu as pltpu
from jax.experimental.pallas import tpu_sc as plsc

NC, NS = 2, 16
NW = NC * NS
N_NODES = 10000
E_EDGES = 320000
CH = 200
NCT = E_EDGES // (NS * CH)
DCH, DNCT = 80, E_EDGES // (NW * 80)
N_PAD = 10112
ROWS_PER_TILE = N_PAD // NS


def _seg_body(src_hbm, dst_hbm, table_hbm, zeros_hbm, out_hbm,
              src_v, dst_v, rows0, rows1, acc, sem0, sem1):
    c = lax.axis_index("c")
    s = lax.axis_index("s")

    pltpu.sync_copy(src_hbm.at[s], src_v)
    pltpu.sync_copy(dst_hbm.at[s], dst_v)

    row0 = s * ROWS_PER_TILE
    pltpu.sync_copy(zeros_hbm.at[pl.ds(row0, ROWS_PER_TILE)],
                    acc.at[pl.ds(row0, ROWS_PER_TILE)])
    plsc.subcore_barrier()

    def gather_start(ci, buf, sem):
        pltpu.async_copy(table_hbm.at[c].at[src_v.at[ci]], buf, sem)

    def gather_wait(ci, buf, sem):
        pltpu.make_async_copy(table_hbm.at[c].at[src_v.at[ci]], buf, sem).wait()

    def scat(ci, buf):
        pltpu.sync_copy(buf, acc.at[dst_v.at[ci]], add=True)

    gather_start(0, rows0, sem0)

    def body(c2, _):
        c0 = 2 * c2
        gather_start(c0 + 1, rows1, sem1)
        gather_wait(c0, rows0, sem0)
        scat(c0, rows0)
        gather_start(c0 + 2, rows0, sem0)
        gather_wait(c0 + 1, rows1, sem1)
        scat(c0 + 1, rows1)
        return 0

    lax.fori_loop(0, (NCT - 2) // 2, body, 0)
    gather_start(NCT - 1, rows1, sem1)
    gather_wait(NCT - 2, rows0, sem0)
    scat(NCT - 2, rows0)
    gather_wait(NCT - 1, rows1, sem1)
    scat(NCT - 1, rows1)

    plsc.subcore_barrier()
    pltpu.sync_copy(acc.at[pl.ds(row0, ROWS_PER_TILE)],
                    out_hbm.at[c, pl.ds(row0, ROWS_PER_TILE)])


def _seg_halves(src3d, dst3d, table, zeros):
    mesh = plsc.VectorSubcoreMesh(core_axis_name="c", subcore_axis_name="s")
    f = pl.kernel(
        _seg_body,
        out_type=jax.ShapeDtypeStruct((NC, N_PAD, 64), jnp.float32),
        mesh=mesh,
        compiler_params=pltpu.CompilerParams(use_tc_tiling_on_sc=False),
        scratch_types=[
            pltpu.VMEM((NCT, CH), jnp.int32),
            pltpu.VMEM((NCT, CH), jnp.int32),
            pltpu.VMEM((CH, 64), jnp.float32),
            pltpu.VMEM((CH, 64), jnp.float32),
            pltpu.VMEM_SHARED((N_PAD, 64), jnp.float32),
            pltpu.SemaphoreType.DMA,
            pltpu.SemaphoreType.DMA,
        ],
    )
    return f(src3d, dst3d, table, zeros)


def _deg_body(dst_hbm, ones_hbm, zeros_hbm, out_hbm, dst_v, ones_v, dacc):
    c = lax.axis_index("c")
    s = lax.axis_index("s")
    wid = s * NC + c

    pltpu.sync_copy(dst_hbm.at[wid], dst_v)
    pltpu.sync_copy(ones_hbm, ones_v)

    row0 = s * ROWS_PER_TILE
    pltpu.sync_copy(zeros_hbm.at[pl.ds(row0, ROWS_PER_TILE)],
                    dacc.at[pl.ds(row0, ROWS_PER_TILE)])
    plsc.subcore_barrier()

    def body(ci, _):
        pltpu.sync_copy(ones_v, dacc.at[dst_v.at[ci]], add=True)
        return 0
    lax.fori_loop(0, DNCT, body, 0)

    plsc.subcore_barrier()
    pltpu.sync_copy(dacc.at[pl.ds(row0, ROWS_PER_TILE)],
                    out_hbm.at[c, pl.ds(row0, ROWS_PER_TILE)])


def _deg_partial(dst2d, ones, zeros16):
    mesh = plsc.VectorSubcoreMesh(core_axis_name="c", subcore_axis_name="s")
    f = pl.kernel(
        _deg_body,
        out_type=jax.ShapeDtypeStruct((NC, N_PAD, 16), jnp.float32),
        mesh=mesh,
        compiler_params=pltpu.CompilerParams(use_tc_tiling_on_sc=False),
        scratch_types=[
            pltpu.VMEM((DNCT, DCH), jnp.int32),
            pltpu.VMEM((DCH, 16), jnp.float32),
            pltpu.VMEM_SHARED((N_PAD, 16), jnp.float32),
        ],
    )
    return f(dst2d, ones, zeros16)


def _se(h, w1t, w2t):
    y = jnp.mean(h, axis=0, keepdims=True)
    a = jnp.maximum(jnp.dot(y, w1t, preferred_element_type=jnp.float32), 0.0)
    sg = jax.nn.sigmoid(jnp.dot(a, w2t, preferred_element_type=jnp.float32))
    return h * sg


def _bn(z, g, be):
    m = jnp.mean(z, axis=0, keepdims=True)
    v = jnp.mean((z - m) ** 2, axis=0, keepdims=True)
    return (z - m) / jnp.sqrt(v + 1e-5) * g + be


def _inproj_body(x_ref, w_ref, b_ref, g_ref, be_ref, tab_ref):
    z = jnp.dot(x_ref[...], w_ref[...], preferred_element_type=jnp.float32)
    z = z + b_ref[...]
    h = jnp.maximum(_bn(z, g_ref[...], be_ref[...]), 0.0)
    tab_ref[0] = h[:, :64]
    tab_ref[1] = h[:, 64:]


def _layer0_body(agg_ref, degp_ref, tab_ref, wla_ref, wlb_ref, wra_ref,
                 wrb_ref, bl_ref, g_ref, be_ref, w1_ref, w2_ref,
                 taba_ref, tabb_ref):
    degp = degp_ref[...]
    deg = jnp.maximum(degp[0, :N_NODES, 0:1] + degp[1, :N_NODES, 0:1], 1.0)
    agg = agg_ref[...]
    tab = tab_ref[...]
    z = (jnp.dot(agg[0, :N_NODES] / deg, wla_ref[...], preferred_element_type=jnp.float32)
         + jnp.dot(agg[1, :N_NODES] / deg, wlb_ref[...], preferred_element_type=jnp.float32)
         + bl_ref[...]
         + jnp.dot(tab[0], wra_ref[...], preferred_element_type=jnp.float32)
         + jnp.dot(tab[1], wrb_ref[...], preferred_element_type=jnp.float32))
    zn = _bn(z, g_ref[...], be_ref[...])
    h = jnp.maximum(_se(zn, w1_ref[...], w2_ref[...]), 0.0)
    taba_ref[0] = h[:, 0:64]
    taba_ref[1] = h[:, 64:128]
    tabb_ref[0] = h[:, 128:192]
    tabb_ref[1] = h[:, 192:256]


def _mean_body(agg_ref, degp_ref, mean_ref):
    degp = degp_ref[...]
    deg = jnp.maximum(degp[0, :N_NODES, 0:1] + degp[1, :N_NODES, 0:1], 1.0)
    agg = agg_ref[...]
    mean_ref[:, :64] = agg[0, :N_NODES] / deg
    mean_ref[:, 64:] = agg[1, :N_NODES] / deg


def _layer1_body(ma_ref, mb_ref, taba_ref, tabb_ref, wla_ref, wlb_ref,
                 wra_ref, wrb_ref, bl_ref, g_ref, be_ref, w1_ref, w2_ref,
                 h2a_ref, h2b_ref):
    taba = taba_ref[...]
    tabb = tabb_ref[...]
    ra = jnp.concatenate([taba[0], taba[1]], axis=1)
    rb = jnp.concatenate([tabb[0], tabb[1]], axis=1)
    z = (jnp.dot(ma_ref[...], wla_ref[...], preferred_element_type=jnp.float32)
         + jnp.dot(mb_ref[...], wlb_ref[...], preferred_element_type=jnp.float32)
         + bl_ref[...]
         + jnp.dot(ra, wra_ref[...], preferred_element_type=jnp.float32)
         + jnp.dot(rb, wrb_ref[...], preferred_element_type=jnp.float32))
    zn = _bn(z, g_ref[...], be_ref[...])
    h = jnp.maximum(_se(zn, w1_ref[...], w2_ref[...]), 0.0)
    h2a_ref[...] = h[:, :128] + ra
    h2b_ref[...] = h[:, 128:] + rb


def _pq_body(h2a_ref, h2b_ref, wl2_ref, wr2_ref, ptab_ref, q_ref):
    h2a = h2a_ref[...]
    h2b = h2b_ref[...]
    p = (jnp.dot(h2a, wl2_ref[:128], preferred_element_type=jnp.float32)
         + jnp.dot(h2b, wl2_ref[128:], preferred_element_type=jnp.float32))
    ptab_ref[0] = p[:, :64]
    ptab_ref[1] = p[:, 64:]
    q_ref[...] = (jnp.dot(h2a, wr2_ref[:128], preferred_element_type=jnp.float32)
                  + jnp.dot(h2b, wr2_ref[128:], preferred_element_type=jnp.float32))


def _final_body(agg_ref, degp_ref, q_ref, bl_ref, g_ref, be_ref,
                wo_ref, bo_ref, out_ref):
    degp = degp_ref[...]
    deg = jnp.maximum(degp[0, :N_NODES, 0:1] + degp[1, :N_NODES, 0:1], 1.0)
    agg = agg_ref[...]
    mean = jnp.concatenate([agg[0, :N_NODES], agg[1, :N_NODES]], axis=1) / deg
    z = mean + bl_ref[...] + q_ref[...]
    h = jnp.maximum(_bn(z, g_ref[...], be_ref[...]), 0.0)
    out_ref[...] = (jnp.dot(h, wo_ref[...], preferred_element_type=jnp.float32)
                    + bo_ref[...])


def _tc(body, out_shape, *args):
    return pl.pallas_call(
        body, out_shape=out_shape,
        compiler_params=pltpu.CompilerParams(vmem_limit_bytes=100 * 1024 * 1024),
    )(*args)


def kernel(x, edge_index, W_in, b_in, g_in, be_in, Wl0, Wr0, bl0, g0, be0,
           se0_W1, se0_W2, Wl1, Wr1, bl1, g1, be1, se1_W1, se1_W2,
           Wl2, Wr2, bl2, g_f, be_f, W_out, b_out):
    f32 = jnp.float32
    src3d = edge_index[0].reshape(NS, NCT, CH)
    dst3d = edge_index[1].reshape(NS, NCT, CH)
    dst_deg = edge_index[1].reshape(NW, DNCT, DCH)
    zeros = jnp.zeros((N_PAD, 64), f32)
    zeros16 = jnp.zeros((N_PAD, 16), f32)

    r1 = lambda a: a.reshape(1, -1)
    sds = jax.ShapeDtypeStruct
    tab_t = sds((NC, N_NODES, 64), f32)

    wl0t, wr0t = Wl0.T, Wr0.T
    wl1t, wr1t = Wl1.T, Wr1.T

    h0tab = _tc(_inproj_body, tab_t, x, W_in.T, r1(b_in), r1(g_in), r1(be_in))

    ones = jnp.ones((DCH, 16), f32)
    degp = _deg_partial(dst_deg, ones, zeros16)
    agg0 = _seg_halves(src3d, dst3d, h0tab, zeros)

    h1taba, h1tabb = _tc(
        _layer0_body, [tab_t, tab_t],
        agg0, degp, h0tab, wl0t[:64], wl0t[64:], wr0t[:64], wr0t[64:],
        r1(bl0), r1(g0), r1(be0), se0_W1.T, se0_W2.T)

    agg1a = _seg_halves(src3d, dst3d, h1taba, zeros)
    agg1b = _seg_halves(src3d, dst3d, h1tabb, zeros)
    m1a = _tc(_mean_body, sds((N_NODES, 128), f32), agg1a, degp)
    m1b = _tc(_mean_body, sds((N_NODES, 128), f32), agg1b, degp)

    h2a, h2b = _tc(
        _layer1_body, [sds((N_NODES, 128), f32), sds((N_NODES, 128), f32)],
        m1a, m1b, h1taba, h1tabb, wl1t[:128], wl1t[128:], wr1t[:128],
        wr1t[128:], r1(bl1), r1(g1), r1(be1), se1_W1.T, se1_W2.T)
    ptab, q = _tc(_pq_body, [tab_t, sds((N_NODES, 128), f32)],
                  h2a, h2b, Wl2.T, Wr2.T)

    agg2 = _seg_halves(src3d, dst3d, ptab, zeros)

    out = _tc(_final_body, sds((N_NODES, 128), f32),
              agg2, degp, q, r1(bl2), r1(g_f), r1(be_f), W_out.T, r1(b_out))
    return out
